# Initial kernel scaffold; baseline (speedup 1.0000x reference)
import jax
import jax.numpy as jnp
from jax.experimental import pallas as pl


def kernel(x, edge_index, edge_attr, We, Wni, Wno, eW1, eW2, n1W1, n1W2, n2W1, n2W2, be, eb1, eb2, n1b1, n1b2, n2b1, n2b2, eps_e, eps_n):
    raise NotImplementedError("write your pallas kernel here")


# speedup vs baseline: 2.5518x; 2.5518x over previous
import functools

import jax
import jax.numpy as jnp
from jax import lax
from jax.experimental import pallas as pl
from jax.experimental.pallas import tpu as pltpu
from jax.experimental.pallas import tpu_sc as plsc

N_NODES = 10000
N_EDGES = 320000
H = 128
DEPTH = 3

NC = 2
NS = 16
NW = NC * NS
EPW = N_EDGES // NW
CH = 80
NCHUNK = EPW // CH
N_PAD = 10112
RPS = N_PAD // NS

EBLK = 2000
NBLK = 2000

_mesh = plsc.VectorSubcoreMesh(core_axis_name="c", subcore_axis_name="s")


@functools.partial(
    pl.kernel,
    out_type=jax.ShapeDtypeStruct((N_EDGES, H), jnp.float32),
    mesh=_mesh,
    scratch_types=[
        pltpu.VMEM((CH,), jnp.int32),
        pltpu.VMEM((CH,), jnp.int32),
        pltpu.VMEM((CH, H), jnp.float32),
        pltpu.SemaphoreType.DMA,
    ],
)
def _sc_gather(xni_hbm, xno_hbm, row_hbm, col_hbm, out_hbm,
               ridx_v, cidx_v, buf_v, sem):
    wid = lax.axis_index("s") * NC + lax.axis_index("c")
    base = wid * EPW

    def body(j, _):
        off = base + j * CH
        pltpu.sync_copy(row_hbm.at[pl.ds(off, CH)], ridx_v)
        pltpu.sync_copy(col_hbm.at[pl.ds(off, CH)], cidx_v)
        pltpu.async_copy(xni_hbm.at[ridx_v], buf_v, sem).wait()
        pltpu.async_copy(xno_hbm.at[cidx_v], buf_v, sem, add=True).wait()
        pltpu.sync_copy(buf_v, out_hbm.at[pl.ds(off, CH)])
        return 0

    lax.fori_loop(0, NCHUNK, body, 0)


@functools.partial(
    pl.kernel,
    out_type=jax.ShapeDtypeStruct((NC, N_PAD, H), jnp.float32),
    mesh=_mesh,
    scratch_types=[
        pltpu.VMEM((CH,), jnp.int32),
        pltpu.VMEM((CH, H), jnp.float32),
        pltpu.VMEM_SHARED((N_PAD, H), jnp.float32),
    ],
)
def _sc_scatter(m_hbm, col_hbm, zero_hbm, out_hbm, cidx_v, buf_v, acc_sh):
    cid = lax.axis_index("c")
    sid = lax.axis_index("s")
    wid = sid * NC + cid
    pltpu.sync_copy(zero_hbm.at[pl.ds(sid * RPS, RPS)],
                    acc_sh.at[pl.ds(sid * RPS, RPS)])
    plsc.subcore_barrier()
    base = wid * EPW

    def body(j, _):
        off = base + j * CH
        pltpu.sync_copy(col_hbm.at[pl.ds(off, CH)], cidx_v)
        pltpu.sync_copy(m_hbm.at[pl.ds(off, CH)], buf_v)
        pltpu.sync_copy(buf_v, acc_sh.at[cidx_v], add=True)
        return 0

    lax.fori_loop(0, NCHUNK, body, 0)
    plsc.subcore_barrier()
    pltpu.sync_copy(acc_sh.at[pl.ds(sid * RPS, RPS)],
                    out_hbm.at[cid, pl.ds(sid * RPS, RPS)])


def _dot(a, b):
    return jnp.dot(a, b, preferred_element_type=jnp.float32)


def _edge_body(scale_ref, edge_ref, fsum_ref, We_ref, eW1_ref, eW2_ref,
               n1W1_ref, n1W2_ref, be_ref, eb1_ref, eb2_ref, n1b1_ref,
               n1b2_ref, newe_ref, m_ref):
    e = edge_ref[...]
    h = jnp.maximum(_dot(e, We_ref[...]) + be_ref[...] + fsum_ref[...], 0.0)
    t = jnp.maximum(_dot(h, eW1_ref[...]) + eb1_ref[...], 0.0)
    ne = scale_ref[0] * e + _dot(t, eW2_ref[...]) + eb2_ref[...]
    u = jnp.maximum(_dot(ne, n1W1_ref[...]) + n1b1_ref[...], 0.0)
    newe_ref[...] = ne
    m_ref[...] = _dot(u, n1W2_ref[...]) + n1b2_ref[...]


_W_SPEC = pl.BlockSpec((H, H), lambda i: (0, 0))
_B_SPEC = pl.BlockSpec((1, H), lambda i: (0, 0))
_S_SPEC = pl.BlockSpec(memory_space=pltpu.SMEM)


_edge_call = pl.pallas_call(
    _edge_body,
    grid=(N_EDGES // EBLK,),
    in_specs=[
        _S_SPEC,
        pl.BlockSpec((EBLK, H), lambda i: (i, 0)),
        pl.BlockSpec((EBLK, H), lambda i: (i, 0)),
        _W_SPEC, _W_SPEC, _W_SPEC, _W_SPEC, _W_SPEC,
        _B_SPEC, _B_SPEC, _B_SPEC, _B_SPEC, _B_SPEC,
    ],
    out_specs=[
        pl.BlockSpec((EBLK, H), lambda i: (i, 0)),
        pl.BlockSpec((EBLK, H), lambda i: (i, 0)),
    ],
    out_shape=[
        jax.ShapeDtypeStruct((N_EDGES, H), jnp.float32),
        jax.ShapeDtypeStruct((N_EDGES, H), jnp.float32),
    ],
)


def _node_body_proj(scale_ref, agg_ref, x_ref, n2W1_ref, n2W2_ref, Wni_ref,
                    Wno_ref, n2b1_ref, n2b2_ref, xnew_ref, xni_ref, xno_ref):
    agg = agg_ref[0] + agg_ref[1]
    t = jnp.maximum(_dot(agg, n2W1_ref[...]) + n2b1_ref[...], 0.0)
    xn = scale_ref[0] * x_ref[...] + _dot(t, n2W2_ref[...]) + n2b2_ref[...]
    xnew_ref[...] = xn
    xni_ref[...] = _dot(xn, Wni_ref[...])
    xno_ref[...] = _dot(xn, Wno_ref[...])


def _node_body_last(scale_ref, agg_ref, x_ref, n2W1_ref, n2W2_ref, n2b1_ref,
                    n2b2_ref, xnew_ref):
    agg = agg_ref[0] + agg_ref[1]
    t = jnp.maximum(_dot(agg, n2W1_ref[...]) + n2b1_ref[...], 0.0)
    xnew_ref[...] = scale_ref[0] * x_ref[...] + _dot(t, n2W2_ref[...]) \
        + n2b2_ref[...]


_AGG_SPEC = pl.BlockSpec((NC, NBLK, H), lambda i: (0, i, 0))
_N_SPEC = pl.BlockSpec((NBLK, H), lambda i: (i, 0))
_NODE_SHAPE = jax.ShapeDtypeStruct((N_NODES, H), jnp.float32)

_node_call_proj = pl.pallas_call(
    _node_body_proj,
    grid=(N_NODES // NBLK,),
    in_specs=[_S_SPEC, _AGG_SPEC, _N_SPEC,
              _W_SPEC, _W_SPEC, _W_SPEC, _W_SPEC, _B_SPEC, _B_SPEC],
    out_specs=[_N_SPEC, _N_SPEC, _N_SPEC],
    out_shape=[_NODE_SHAPE, _NODE_SHAPE, _NODE_SHAPE],
)

_node_call_last = pl.pallas_call(
    _node_body_last,
    grid=(N_NODES // NBLK,),
    in_specs=[_S_SPEC, _AGG_SPEC, _N_SPEC, _W_SPEC, _W_SPEC, _B_SPEC,
              _B_SPEC],
    out_specs=_N_SPEC,
    out_shape=_NODE_SHAPE,
)


def _proj_body(x_ref, Wni_ref, Wno_ref, xni_ref, xno_ref):
    x = x_ref[...]
    xni_ref[...] = _dot(x, Wni_ref[...])
    xno_ref[...] = _dot(x, Wno_ref[...])


_proj_call = pl.pallas_call(
    _proj_body,
    grid=(N_NODES // NBLK,),
    in_specs=[_N_SPEC, _W_SPEC, _W_SPEC],
    out_specs=[_N_SPEC, _N_SPEC],
    out_shape=[_NODE_SHAPE, _NODE_SHAPE],
)


def kernel(x, edge_index, edge_attr, We, Wni, Wno, eW1, eW2, n1W1, n1W2,
           n2W1, n2W2, be, eb1, eb2, n1b1, n1b2, n2b1, n2b2, eps_e, eps_n):
    row = edge_index[0]
    col = edge_index[1]
    zeros_nh = jnp.zeros((N_PAD, H), jnp.float32)

    xni, xno = _proj_call(x, Wni[0], Wno[0])
    for i in range(DEPTH):
        fsum = _sc_gather(xni, xno, row, col)
        scale_e = (1.0 + eps_e[i]).reshape((1,))
        edge_attr, m = _edge_call(
            scale_e, edge_attr, fsum, We[i], eW1[i], eW2[i], n1W1[i],
            n1W2[i], be[i].reshape(1, H), eb1[i].reshape(1, H),
            eb2[i].reshape(1, H), n1b1[i].reshape(1, H),
            n1b2[i].reshape(1, H))
        agg2 = _sc_scatter(m, col, zeros_nh)
        scale_n = (1.0 + eps_n[i]).reshape((1,))
        if i + 1 < DEPTH:
            x, xni, xno = _node_call_proj(
                scale_n, agg2, x, n2W1[i], n2W2[i], Wni[i + 1], Wno[i + 1],
                n2b1[i].reshape(1, H), n2b2[i].reshape(1, H))
        else:
            x = _node_call_last(
                scale_n, agg2, x, n2W1[i], n2W2[i],
                n2b1[i].reshape(1, H), n2b2[i].reshape(1, H))
    return (x, edge_attr)


# speedup vs baseline: 3.9796x; 1.5595x over previous
import functools

import jax
import jax.numpy as jnp
from jax import lax
from jax.experimental import pallas as pl
from jax.experimental.pallas import tpu as pltpu
from jax.experimental.pallas import tpu_sc as plsc

N_NODES = 10000
N_EDGES = 320000
H = 128
DEPTH = 3

NC = 2
NS = 16
NW = NC * NS
EPW = N_EDGES // NW
CH = 80
NCHUNK = EPW // CH
N_PAD = 10112
RPS = N_PAD // NS

EBLK = 2000
NBLK = 2000

_mesh = plsc.VectorSubcoreMesh(core_axis_name="c", subcore_axis_name="s")


K = 5
OUTER = NCHUNK // K
CHS = 40
NCHUNKS = EPW // CHS
OUTERS = NCHUNKS // K


@functools.partial(
    pl.kernel,
    out_type=jax.ShapeDtypeStruct((N_EDGES, H), jnp.float32),
    mesh=_mesh,
    scratch_types=[
        pltpu.VMEM((EPW,), jnp.int32),
        pltpu.VMEM((EPW,), jnp.int32),
        pltpu.VMEM((K, CH, H), jnp.float32),
        pltpu.SemaphoreType.DMA,
        pltpu.SemaphoreType.DMA,
        pltpu.SemaphoreType.DMA,
    ],
)
def _sc_gather(xni_hbm, xno_hbm, row_hbm, col_hbm, out_hbm,
               ridx_v, cidx_v, bufs, sem_g, sem_a, sem_w):
    wid = lax.axis_index("s") * NC + lax.axis_index("c")
    base = wid * EPW
    pltpu.sync_copy(row_hbm.at[pl.ds(base, EPW)], ridx_v)
    pltpu.sync_copy(col_hbm.at[pl.ds(base, EPW)], cidx_v)

    def outer(g, _):
        @pl.when(g > 0)
        def _():
            for b in range(K):
                off = base + ((g - 1) * K + b) * CH
                pltpu.make_async_copy(
                    bufs.at[b], out_hbm.at[pl.ds(off, CH)], sem_w).wait()
        ds = []
        for b in range(K):
            c = (g * K + b) * CH
            ds.append(pltpu.async_copy(
                xni_hbm.at[ridx_v.at[pl.ds(c, CH)]], bufs.at[b], sem_g))
        for d in ds:
            d.wait()
        ds = []
        for b in range(K):
            c = (g * K + b) * CH
            ds.append(pltpu.async_copy(
                xno_hbm.at[cidx_v.at[pl.ds(c, CH)]], bufs.at[b], sem_a,
                add=True))
        for d in ds:
            d.wait()
        for b in range(K):
            off = base + (g * K + b) * CH
            pltpu.async_copy(bufs.at[b], out_hbm.at[pl.ds(off, CH)], sem_w)
        return 0

    lax.fori_loop(0, OUTER, outer, 0)
    for b in range(K):
        off = base + ((OUTER - 1) * K + b) * CH
        pltpu.make_async_copy(
            bufs.at[b], out_hbm.at[pl.ds(off, CH)], sem_w).wait()


@functools.partial(
    pl.kernel,
    out_type=jax.ShapeDtypeStruct((NC, N_PAD, H), jnp.float32),
    mesh=_mesh,
    scratch_types=[
        pltpu.VMEM((K, CHS), jnp.int32),
        pltpu.VMEM((K, CHS, H), jnp.float32),
        pltpu.VMEM_SHARED((N_PAD, H), jnp.float32),
        pltpu.SemaphoreType.DMA,
        pltpu.SemaphoreType.DMA,
        pltpu.SemaphoreType.DMA,
    ],
)
def _sc_scatter(m_hbm, col_hbm, zero_hbm, out_hbm, cidx2, bufs, acc_sh,
                sem_i, sem_m, sem_s):
    cid = lax.axis_index("c")
    sid = lax.axis_index("s")
    wid = sid * NC + cid
    pltpu.sync_copy(zero_hbm.at[pl.ds(sid * RPS, RPS)],
                    acc_sh.at[pl.ds(sid * RPS, RPS)])
    plsc.subcore_barrier()
    base = wid * EPW

    def outer(g, _):
        @pl.when(g > 0)
        def _():
            for b in range(K):
                pltpu.make_async_copy(
                    bufs.at[b], acc_sh.at[cidx2.at[b]], sem_s).wait()
        ds = []
        for b in range(K):
            off = base + (g * K + b) * CHS
            ds.append(pltpu.async_copy(
                col_hbm.at[pl.ds(off, CHS)], cidx2.at[b], sem_i))
            ds.append(pltpu.async_copy(
                m_hbm.at[pl.ds(off, CHS)], bufs.at[b], sem_m))
        for d in ds:
            d.wait()
        for b in range(K):
            pltpu.async_copy(bufs.at[b], acc_sh.at[cidx2.at[b]], sem_s,
                             add=True)
        return 0

    lax.fori_loop(0, OUTERS, outer, 0)
    for b in range(K):
        pltpu.make_async_copy(bufs.at[b], acc_sh.at[cidx2.at[b]], sem_s).wait()
    plsc.subcore_barrier()
    pltpu.sync_copy(acc_sh.at[pl.ds(sid * RPS, RPS)],
                    out_hbm.at[cid, pl.ds(sid * RPS, RPS)])


def _dot(a, b):
    return jnp.dot(a, b, preferred_element_type=jnp.float32)


def _edge_body(scale_ref, edge_ref, fsum_ref, We_ref, eW1_ref, eW2_ref,
               n1W1_ref, n1W2_ref, be_ref, eb1_ref, eb2_ref, n1b1_ref,
               n1b2_ref, newe_ref, m_ref):
    e = edge_ref[...]
    h = jnp.maximum(_dot(e, We_ref[...]) + be_ref[...] + fsum_ref[...], 0.0)
    t = jnp.maximum(_dot(h, eW1_ref[...]) + eb1_ref[...], 0.0)
    ne = scale_ref[0] * e + _dot(t, eW2_ref[...]) + eb2_ref[...]
    u = jnp.maximum(_dot(ne, n1W1_ref[...]) + n1b1_ref[...], 0.0)
    newe_ref[...] = ne
    m_ref[...] = _dot(u, n1W2_ref[...]) + n1b2_ref[...]


_W_SPEC = pl.BlockSpec((H, H), lambda i: (0, 0))
_B_SPEC = pl.BlockSpec((1, H), lambda i: (0, 0))
_S_SPEC = pl.BlockSpec(memory_space=pltpu.SMEM)


_edge_call = pl.pallas_call(
    _edge_body,
    grid=(N_EDGES // EBLK,),
    in_specs=[
        _S_SPEC,
        pl.BlockSpec((EBLK, H), lambda i: (i, 0)),
        pl.BlockSpec((EBLK, H), lambda i: (i, 0)),
        _W_SPEC, _W_SPEC, _W_SPEC, _W_SPEC, _W_SPEC,
        _B_SPEC, _B_SPEC, _B_SPEC, _B_SPEC, _B_SPEC,
    ],
    out_specs=[
        pl.BlockSpec((EBLK, H), lambda i: (i, 0)),
        pl.BlockSpec((EBLK, H), lambda i: (i, 0)),
    ],
    out_shape=[
        jax.ShapeDtypeStruct((N_EDGES, H), jnp.float32),
        jax.ShapeDtypeStruct((N_EDGES, H), jnp.float32),
    ],
)


def _node_body_proj(scale_ref, agg_ref, x_ref, n2W1_ref, n2W2_ref, Wni_ref,
                    Wno_ref, n2b1_ref, n2b2_ref, xnew_ref, xni_ref, xno_ref):
    agg = agg_ref[0] + agg_ref[1]
    t = jnp.maximum(_dot(agg, n2W1_ref[...]) + n2b1_ref[...], 0.0)
    xn = scale_ref[0] * x_ref[...] + _dot(t, n2W2_ref[...]) + n2b2_ref[...]
    xnew_ref[...] = xn
    xni_ref[...] = _dot(xn, Wni_ref[...])
    xno_ref[...] = _dot(xn, Wno_ref[...])


def _node_body_last(scale_ref, agg_ref, x_ref, n2W1_ref, n2W2_ref, n2b1_ref,
                    n2b2_ref, xnew_ref):
    agg = agg_ref[0] + agg_ref[1]
    t = jnp.maximum(_dot(agg, n2W1_ref[...]) + n2b1_ref[...], 0.0)
    xnew_ref[...] = scale_ref[0] * x_ref[...] + _dot(t, n2W2_ref[...]) \
        + n2b2_ref[...]


_AGG_SPEC = pl.BlockSpec((NC, NBLK, H), lambda i: (0, i, 0))
_N_SPEC = pl.BlockSpec((NBLK, H), lambda i: (i, 0))
_NODE_SHAPE = jax.ShapeDtypeStruct((N_NODES, H), jnp.float32)

_node_call_proj = pl.pallas_call(
    _node_body_proj,
    grid=(N_NODES // NBLK,),
    in_specs=[_S_SPEC, _AGG_SPEC, _N_SPEC,
              _W_SPEC, _W_SPEC, _W_SPEC, _W_SPEC, _B_SPEC, _B_SPEC],
    out_specs=[_N_SPEC, _N_SPEC, _N_SPEC],
    out_shape=[_NODE_SHAPE, _NODE_SHAPE, _NODE_SHAPE],
)

_node_call_last = pl.pallas_call(
    _node_body_last,
    grid=(N_NODES // NBLK,),
    in_specs=[_S_SPEC, _AGG_SPEC, _N_SPEC, _W_SPEC, _W_SPEC, _B_SPEC,
              _B_SPEC],
    out_specs=_N_SPEC,
    out_shape=_NODE_SHAPE,
)


def _proj_body(x_ref, Wni_ref, Wno_ref, xni_ref, xno_ref):
    x = x_ref[...]
    xni_ref[...] = _dot(x, Wni_ref[...])
    xno_ref[...] = _dot(x, Wno_ref[...])


_proj_call = pl.pallas_call(
    _proj_body,
    grid=(N_NODES // NBLK,),
    in_specs=[_N_SPEC, _W_SPEC, _W_SPEC],
    out_specs=[_N_SPEC, _N_SPEC],
    out_shape=[_NODE_SHAPE, _NODE_SHAPE],
)


def kernel(x, edge_index, edge_attr, We, Wni, Wno, eW1, eW2, n1W1, n1W2,
           n2W1, n2W2, be, eb1, eb2, n1b1, n1b2, n2b1, n2b2, eps_e, eps_n):
    row = edge_index[0]
    col = edge_index[1]
    zeros_nh = jnp.zeros((N_PAD, H), jnp.float32)

    xni, xno = _proj_call(x, Wni[0], Wno[0])
    for i in range(DEPTH):
        fsum = _sc_gather(xni, xno, row, col)
        scale_e = (1.0 + eps_e[i]).reshape((1,))
        edge_attr, m = _edge_call(
            scale_e, edge_attr, fsum, We[i], eW1[i], eW2[i], n1W1[i],
            n1W2[i], be[i].reshape(1, H), eb1[i].reshape(1, H),
            eb2[i].reshape(1, H), n1b1[i].reshape(1, H),
            n1b2[i].reshape(1, H))
        agg2 = _sc_scatter(m, col, zeros_nh)
        scale_n = (1.0 + eps_n[i]).reshape((1,))
        if i + 1 < DEPTH:
            x, xni, xno = _node_call_proj(
                scale_n, agg2, x, n2W1[i], n2W2[i], Wni[i + 1], Wno[i + 1],
                n2b1[i].reshape(1, H), n2b2[i].reshape(1, H))
        else:
            x = _node_call_last(
                scale_n, agg2, x, n2W1[i], n2W2[i],
                n2b1[i].reshape(1, H), n2b2[i].reshape(1, H))
    return (x, edge_attr)
